# Initial kernel scaffold; baseline (speedup 1.0000x reference)
import jax
import jax.numpy as jnp
from jax.experimental import pallas as pl


def kernel(features, edge_index_e0, edge_index_e1, edge_index_e2, W_e0, b_e0, W_e1, b_e1, W_e2, b_e2):
    raise NotImplementedError("write your pallas kernel here")


# speedup vs baseline: 3.7079x; 3.7079x over previous
import functools

import jax
import jax.numpy as jnp
from jax import lax
from jax.experimental import pallas as pl
from jax.experimental.pallas import tpu as pltpu
from jax.experimental.pallas import tpu_sc as plsc

_N = 10000
_E = 320000
_D = 128
_NC = 2
_NS = 16
_NW = _NC * _NS
_EPT = _E // _NW
_K = 80
_NCHUNK = _EPT // _K
_NP = 10240
_RPT = _NP // _NS
_ZR = 8


def _matmul_body(f_ref, w0, b0, w1, b1, w2, b2, o0, o1, o2):
    f = f_ref[...]
    o0[...] = jnp.dot(f, w0[...], preferred_element_type=jnp.float32) + b0[...]
    o1[...] = jnp.dot(f, w1[...], preferred_element_type=jnp.float32) + b1[...]
    o2[...] = jnp.dot(f, w2[...], preferred_element_type=jnp.float32) + b2[...]


def _matmul(features, W0, b0, W1, b1, W2, b2):
    BM = 2000
    grid = (_N // BM,)
    wspec = pl.BlockSpec((_D, _D), lambda i: (0, 0))
    bspec = pl.BlockSpec((1, _D), lambda i: (0, 0))
    fspec = pl.BlockSpec((BM, _D), lambda i: (i, 0))
    return pl.pallas_call(
        _matmul_body,
        grid=grid,
        in_specs=[fspec, wspec, bspec, wspec, bspec, wspec, bspec],
        out_specs=[fspec, fspec, fspec],
        out_shape=[jax.ShapeDtypeStruct((_N, _D), jnp.float32)] * 3,
    )(features, W0, b0.reshape(1, _D), W1, b1.reshape(1, _D), W2, b2.reshape(1, _D))


def _agg_body(src0, dst0, src1, dst1, src2, dst2, wh0, wh1, wh2,
              psum,
              acc, idx_s, idx_d, rows, ones_b, ztmp, sem):
    c = lax.axis_index("c")
    s = lax.axis_index("s")
    wid = c * _NS + s
    base_r = s * _RPT
    ebase = wid * _EPT

    def _fill_z(r, carry):
        for j in range(_D // 16):
            ztmp[r, pl.ds(16 * j, 16)] = jnp.zeros((16,), jnp.float32)
        return carry

    lax.fori_loop(0, _ZR, _fill_z, 0)

    def _fill_o(r, carry):
        for j in range(_D // 16):
            ones_b[r, pl.ds(16 * j, 16)] = jnp.ones((16,), jnp.float32)
        return carry

    lax.fori_loop(0, _K, _fill_o, 0)

    passes = ((src0, dst0, wh0, True), (src1, dst1, wh1, True),
              (src2, dst2, wh2, True), (src0, dst0, wh0, False),
              (src1, dst1, wh1, False), (src2, dst2, wh2, False))
    for p, (src_ref, dst_ref, wh_ref, is_sum) in enumerate(passes):
        def _zero(t, carry):
            r0 = pl.multiple_of(base_r + t * _ZR, 8)
            pltpu.sync_copy(ztmp, acc.at[pl.ds(r0, _ZR)])
            return carry

        lax.fori_loop(0, _RPT // _ZR, _zero, 0)
        plsc.subcore_barrier()

        if is_sum:
            def _chunk(j, carry):
                off = ebase + j * _K
                pltpu.sync_copy(src_ref.at[pl.ds(off, _K)], idx_s)
                pltpu.sync_copy(dst_ref.at[pl.ds(off, _K)], idx_d)
                pltpu.async_copy(wh_ref.at[idx_s], rows, sem).wait()
                pltpu.sync_copy(rows, acc.at[idx_d], add=True)
                return carry
        else:
            def _chunk(j, carry):
                off = ebase + j * _K
                pltpu.sync_copy(dst_ref.at[pl.ds(off, _K)], idx_d)
                pltpu.sync_copy(ones_b, acc.at[idx_d], add=True)
                return carry

        lax.fori_loop(0, _NCHUNK, _chunk, 0)
        plsc.subcore_barrier()

        def _drain(t, carry):
            r0 = pl.multiple_of(base_r + t * _K, 8)
            pltpu.sync_copy(acc.at[pl.ds(r0, _K)], rows)
            pltpu.sync_copy(rows, psum.at[p, c, pl.ds(r0, _K)])
            return carry

        lax.fori_loop(0, _RPT // _K, _drain, 0)


@functools.partial(jax.jit, static_argnums=())
def _agg(src0, dst0, src1, dst1, src2, dst2, wh0, wh1, wh2):
    mesh = plsc.VectorSubcoreMesh(core_axis_name="c", subcore_axis_name="s",
                                  num_cores=_NC, num_subcores=_NS)
    kern = pl.kernel(
        _agg_body,
        out_type=jax.ShapeDtypeStruct((6, _NC, _NP, _D), jnp.float32),
        mesh=mesh,
        scratch_types=[
            pltpu.VMEM_SHARED((_NP, _D), jnp.float32),
            pltpu.VMEM((_K,), jnp.int32),
            pltpu.VMEM((_K,), jnp.int32),
            pltpu.VMEM((_K, _D), jnp.float32),
            pltpu.VMEM((_K, _D), jnp.float32),
            pltpu.VMEM((_ZR, _D), jnp.float32),
            pltpu.SemaphoreType.DMA,
        ],
    )
    return kern(src0, dst0, src1, dst1, src2, dst2, wh0, wh1, wh2)


def _combine_body(ps_ref, o_ref):
    acc = jnp.zeros(o_ref.shape, jnp.float32)
    for e in range(3):
        sm = ps_ref[e, 0] + ps_ref[e, 1]
        cn = ps_ref[3 + e, 0, :, 0:1] + ps_ref[3 + e, 1, :, 0:1]
        acc = acc + sm / jnp.maximum(cn, 1.0)
    o_ref[...] = jnp.maximum(acc, 0.0)


def _combine(psum):
    BM = 1000
    grid = (_N // BM,)
    return pl.pallas_call(
        _combine_body,
        grid=grid,
        in_specs=[pl.BlockSpec((6, _NC, BM, _D), lambda i: (0, 0, i, 0))],
        out_specs=pl.BlockSpec((BM, _D), lambda i: (i, 0)),
        out_shape=jax.ShapeDtypeStruct((_N, _D), jnp.float32),
    )(psum)


def kernel(features, edge_index_e0, edge_index_e1, edge_index_e2,
           W_e0, b_e0, W_e1, b_e1, W_e2, b_e2):
    wh0, wh1, wh2 = _matmul(features, W_e0, b_e0, W_e1, b_e1, W_e2, b_e2)
    psum = _agg(edge_index_e0[0], edge_index_e0[1],
                edge_index_e1[0], edge_index_e1[1],
                edge_index_e2[0], edge_index_e2[1],
                wh0, wh1, wh2)
    return _combine(psum)


# speedup vs baseline: 5.1084x; 1.3777x over previous
import functools

import jax
import jax.numpy as jnp
from jax import lax
from jax.experimental import pallas as pl
from jax.experimental.pallas import tpu as pltpu
from jax.experimental.pallas import tpu_sc as plsc

_N = 10000
_E = 320000
_D = 128
_NC = 2
_NS = 16
_NW = _NC * _NS
_EPT = _E // _NW
_K = 80
_NCHUNK = _EPT // _K
_NP = 10240
_RPT = _NP // _NS
_ZR = 8


def _matmul_body(f_ref, w0, b0, w1, b1, w2, b2, o0, o1, o2):
    f = f_ref[...]
    o0[...] = jnp.dot(f, w0[...], preferred_element_type=jnp.float32) + b0[...]
    o1[...] = jnp.dot(f, w1[...], preferred_element_type=jnp.float32) + b1[...]
    o2[...] = jnp.dot(f, w2[...], preferred_element_type=jnp.float32) + b2[...]


def _matmul(features, W0, b0, W1, b1, W2, b2):
    BM = 2000
    grid = (_N // BM,)
    wspec = pl.BlockSpec((_D, _D), lambda i: (0, 0))
    bspec = pl.BlockSpec((1, _D), lambda i: (0, 0))
    fspec = pl.BlockSpec((BM, _D), lambda i: (i, 0))
    return pl.pallas_call(
        _matmul_body,
        grid=grid,
        in_specs=[fspec, wspec, bspec, wspec, bspec, wspec, bspec],
        out_specs=[fspec, fspec, fspec],
        out_shape=[jax.ShapeDtypeStruct((_N, _D), jnp.float32)] * 3,
    )(features, W0, b0.reshape(1, _D), W1, b1.reshape(1, _D), W2, b2.reshape(1, _D))


def _agg_body(src0, dst0, src1, dst1, src2, dst2, wh0, wh1, wh2,
              psum,
              acc, idx_s0, idx_s1, idx_d0, idx_d1, rows0, rows1, ztmp,
              sem0, sem1):
    c = lax.axis_index("c")
    s = lax.axis_index("s")
    wid = c * _NS + s
    base_r = s * _RPT
    ebase = wid * _EPT
    idx_s = (idx_s0, idx_s1)
    idx_d = (idx_d0, idx_d1)
    rows = (rows0, rows1)
    gsem = (sem0, sem1)

    def _fill_z(r, carry):
        for j in range(_D // 16):
            ztmp[r, pl.ds(16 * j, 16)] = jnp.zeros((16,), jnp.float32)
        return carry

    lax.fori_loop(0, _ZR, _fill_z, 0)

    def _fill_ones(r, carry):
        for j in range(_D // 16):
            rows1[r, pl.ds(16 * j, 16)] = jnp.ones((16,), jnp.float32)
        return carry

    passes = ((src0, dst0, wh0, True), (src1, dst1, wh1, True),
              (src2, dst2, wh2, True), (src0, dst0, wh0, False),
              (src1, dst1, wh1, False), (src2, dst2, wh2, False))
    for p, (src_ref, dst_ref, wh_ref, is_sum) in enumerate(passes):
        if p == 3:
            lax.fori_loop(0, _K, _fill_ones, 0)

        def _zero(t, carry):
            r0 = pl.multiple_of(base_r + t * _ZR, 8)
            pltpu.sync_copy(ztmp, acc.at[pl.ds(r0, _ZR)])
            return carry

        lax.fori_loop(0, _RPT // _ZR, _zero, 0)
        plsc.subcore_barrier()

        if is_sum:
            for b in range(2):
                off = ebase + b * _K
                pltpu.sync_copy(src_ref.at[pl.ds(off, _K)], idx_s[b])
                pltpu.sync_copy(dst_ref.at[pl.ds(off, _K)], idx_d[b])
                pltpu.async_copy(wh_ref.at[idx_s[b]], rows[b], gsem[b])

            def _pair(jj, carry):
                for b in range(2):
                    pltpu.make_async_copy(wh_ref.at[pl.ds(0, _K)], rows[b],
                                          gsem[b]).wait()
                    pltpu.sync_copy(rows[b], acc.at[idx_d[b]], add=True)
                    off = ebase + (2 * jj + b) * _K
                    pltpu.sync_copy(src_ref.at[pl.ds(off, _K)], idx_s[b])
                    pltpu.sync_copy(dst_ref.at[pl.ds(off, _K)], idx_d[b])
                    pltpu.async_copy(wh_ref.at[idx_s[b]], rows[b], gsem[b])
                return carry

            lax.fori_loop(1, (_NCHUNK - 1) // 2, _pair, 0)
            for b in range(2):
                pltpu.make_async_copy(wh_ref.at[pl.ds(0, _K)], rows[b],
                                      gsem[b]).wait()
                pltpu.sync_copy(rows[b], acc.at[idx_d[b]], add=True)
            off = ebase + (_NCHUNK - 1) * _K
            pltpu.sync_copy(src_ref.at[pl.ds(off, _K)], idx_s0)
            pltpu.sync_copy(dst_ref.at[pl.ds(off, _K)], idx_d0)
            pltpu.async_copy(wh_ref.at[idx_s0], rows0, sem0).wait()
            pltpu.sync_copy(rows0, acc.at[idx_d0], add=True)
        else:
            def _chunk(j, carry):
                off = ebase + j * _K
                pltpu.sync_copy(dst_ref.at[pl.ds(off, _K)], idx_d0)
                pltpu.sync_copy(rows1, acc.at[idx_d0], add=True)
                return carry

            lax.fori_loop(0, _NCHUNK, _chunk, 0)
        plsc.subcore_barrier()

        def _drain(t, carry):
            r0 = pl.multiple_of(base_r + t * _K, 8)
            pltpu.sync_copy(acc.at[pl.ds(r0, _K)], rows0)
            pltpu.sync_copy(rows0, psum.at[p, c, pl.ds(r0, _K)])
            return carry

        lax.fori_loop(0, _RPT // _K, _drain, 0)


@functools.partial(jax.jit, static_argnums=())
def _agg(src0, dst0, src1, dst1, src2, dst2, wh0, wh1, wh2):
    mesh = plsc.VectorSubcoreMesh(core_axis_name="c", subcore_axis_name="s",
                                  num_cores=_NC, num_subcores=_NS)
    kern = pl.kernel(
        _agg_body,
        out_type=jax.ShapeDtypeStruct((6, _NC, _NP, _D), jnp.float32),
        mesh=mesh,
        scratch_types=[
            pltpu.VMEM_SHARED((_NP, _D), jnp.float32),
            pltpu.VMEM((_K,), jnp.int32),
            pltpu.VMEM((_K,), jnp.int32),
            pltpu.VMEM((_K,), jnp.int32),
            pltpu.VMEM((_K,), jnp.int32),
            pltpu.VMEM((_K, _D), jnp.float32),
            pltpu.VMEM((_K, _D), jnp.float32),
            pltpu.VMEM((_ZR, _D), jnp.float32),
            pltpu.SemaphoreType.DMA,
            pltpu.SemaphoreType.DMA,
        ],
    )
    return kern(src0, dst0, src1, dst1, src2, dst2, wh0, wh1, wh2)


def _combine_body(ps_ref, o_ref):
    acc = jnp.zeros(o_ref.shape, jnp.float32)
    for e in range(3):
        sm = ps_ref[e, 0] + ps_ref[e, 1]
        cn = ps_ref[3 + e, 0, :, 0:1] + ps_ref[3 + e, 1, :, 0:1]
        acc = acc + sm / jnp.maximum(cn, 1.0)
    o_ref[...] = jnp.maximum(acc, 0.0)


def _combine(psum):
    BM = 1000
    grid = (_N // BM,)
    return pl.pallas_call(
        _combine_body,
        grid=grid,
        in_specs=[pl.BlockSpec((6, _NC, BM, _D), lambda i: (0, 0, i, 0))],
        out_specs=pl.BlockSpec((BM, _D), lambda i: (i, 0)),
        out_shape=jax.ShapeDtypeStruct((_N, _D), jnp.float32),
    )(psum)


def kernel(features, edge_index_e0, edge_index_e1, edge_index_e2,
           W_e0, b_e0, W_e1, b_e1, W_e2, b_e2):
    wh0, wh1, wh2 = _matmul(features, W_e0, b_e0, W_e1, b_e1, W_e2, b_e2)
    psum = _agg(edge_index_e0[0], edge_index_e0[1],
                edge_index_e1[0], edge_index_e1[1],
                edge_index_e2[0], edge_index_e2[1],
                wh0, wh1, wh2)
    return _combine(psum)


# speedup vs baseline: 6.0234x; 1.1791x over previous
import functools

import jax
import jax.numpy as jnp
from jax import lax
from jax.experimental import pallas as pl
from jax.experimental.pallas import tpu as pltpu
from jax.experimental.pallas import tpu_sc as plsc

_N = 10000
_E = 320000
_D = 128
_NC = 2
_NS = 16
_NW = _NC * _NS
_EPT = _E // _NW
_K = 80
_NCHUNK = _EPT // _K
_NP = 10240
_RPT = _NP // _NS
_ZR = 8


def _matmul_body(f_ref, w0, b0, w1, b1, w2, b2, o0, o1, o2):
    f = f_ref[...]
    o0[...] = jnp.dot(f, w0[...], preferred_element_type=jnp.float32) + b0[...]
    o1[...] = jnp.dot(f, w1[...], preferred_element_type=jnp.float32) + b1[...]
    o2[...] = jnp.dot(f, w2[...], preferred_element_type=jnp.float32) + b2[...]


def _matmul(features, W0, b0, W1, b1, W2, b2):
    BM = 2000
    grid = (_N // BM,)
    wspec = pl.BlockSpec((_D, _D), lambda i: (0, 0))
    bspec = pl.BlockSpec((1, _D), lambda i: (0, 0))
    fspec = pl.BlockSpec((BM, _D), lambda i: (i, 0))
    return pl.pallas_call(
        _matmul_body,
        grid=grid,
        in_specs=[fspec, wspec, bspec, wspec, bspec, wspec, bspec],
        out_specs=[fspec, fspec, fspec],
        out_shape=[jax.ShapeDtypeStruct((_N, _D), jnp.float32)] * 3,
    )(features, W0, b0.reshape(1, _D), W1, b1.reshape(1, _D), W2, b2.reshape(1, _D))


def _agg_body(src0, dst0, src1, dst1, src2, dst2, wh0, wh1, wh2,
              psum,
              acc, idx_s0, idx_s1, idx_d0, idx_d1, rows0, rows1, ztmp,
              sem0, sem1):
    c = lax.axis_index("c")
    s = lax.axis_index("s")
    wid = c * _NS + s
    base_r = s * _RPT
    ebase = wid * _EPT
    idx_s = (idx_s0, idx_s1)
    idx_d = (idx_d0, idx_d1)
    rows = (rows0, rows1)
    gsem = (sem0, sem1)

    def _fill_z(r, carry):
        for j in range(_D // 16):
            ztmp[r, pl.ds(16 * j, 16)] = jnp.zeros((16,), jnp.float32)
        return carry

    lax.fori_loop(0, _ZR, _fill_z, 0)

    def _fill_ones(r, carry):
        for j in range(_D // 16):
            rows1[r, pl.ds(16 * j, 16)] = jnp.ones((16,), jnp.float32)
        return carry

    passes = ((src0, dst0, wh0, True), (src1, dst1, wh1, True),
              (src2, dst2, wh2, True), (src0, dst0, wh0, False),
              (src1, dst1, wh1, False), (src2, dst2, wh2, False))
    for p, (src_ref, dst_ref, wh_ref, is_sum) in enumerate(passes):
        if p == 3:
            lax.fori_loop(0, _K, _fill_ones, 0)

        def _zero(t, carry):
            r0 = pl.multiple_of(base_r + t * _ZR, 8)
            pltpu.sync_copy(ztmp, acc.at[pl.ds(r0, _ZR)])
            return carry

        lax.fori_loop(0, _RPT // _ZR, _zero, 0)
        plsc.subcore_barrier()

        if is_sum:
            for b in range(2):
                off = ebase + b * _K
                pltpu.sync_copy(src_ref.at[pl.ds(off, _K)], idx_s[b])
                pltpu.sync_copy(dst_ref.at[pl.ds(off, _K)], idx_d[b])
                pltpu.async_copy(wh_ref.at[idx_s[b]], rows[b], gsem[b])

            def _pair(jj, carry):
                for b in range(2):
                    pltpu.make_async_copy(wh_ref.at[pl.ds(0, _K)], rows[b],
                                          gsem[b]).wait()
                    pltpu.sync_copy(rows[b], acc.at[idx_d[b]], add=True)
                    off = ebase + (2 * jj + b) * _K
                    pltpu.sync_copy(src_ref.at[pl.ds(off, _K)], idx_s[b])
                    pltpu.sync_copy(dst_ref.at[pl.ds(off, _K)], idx_d[b])
                    pltpu.async_copy(wh_ref.at[idx_s[b]], rows[b], gsem[b])
                return carry

            lax.fori_loop(1, (_NCHUNK - 1) // 2, _pair, 0)
            for b in range(2):
                pltpu.make_async_copy(wh_ref.at[pl.ds(0, _K)], rows[b],
                                      gsem[b]).wait()
                pltpu.sync_copy(rows[b], acc.at[idx_d[b]], add=True)
            off = ebase + (_NCHUNK - 1) * _K
            pltpu.sync_copy(src_ref.at[pl.ds(off, _K)], idx_s0)
            pltpu.sync_copy(dst_ref.at[pl.ds(off, _K)], idx_d0)
            pltpu.async_copy(wh_ref.at[idx_s0], rows0, sem0).wait()
            pltpu.sync_copy(rows0, acc.at[idx_d0], add=True)
        else:
            for b in range(2):
                off = ebase + b * _K
                pltpu.async_copy(dst_ref.at[pl.ds(off, _K)], idx_d[b], gsem[b])

            def _chunk(jj, carry):
                for b in range(2):
                    pltpu.make_async_copy(dst_ref.at[pl.ds(0, _K)], idx_d[b],
                                          gsem[b]).wait()
                    pltpu.sync_copy(rows1, acc.at[idx_d[b]], add=True)
                    off = ebase + (2 * jj + b) * _K
                    pltpu.async_copy(dst_ref.at[pl.ds(off, _K)], idx_d[b],
                                     gsem[b])
                return carry

            lax.fori_loop(1, (_NCHUNK - 1) // 2, _chunk, 0)
            for b in range(2):
                pltpu.make_async_copy(dst_ref.at[pl.ds(0, _K)], idx_d[b],
                                      gsem[b]).wait()
                pltpu.sync_copy(rows1, acc.at[idx_d[b]], add=True)
            off = ebase + (_NCHUNK - 1) * _K
            pltpu.sync_copy(dst_ref.at[pl.ds(off, _K)], idx_d0)
            pltpu.sync_copy(rows1, acc.at[idx_d0], add=True)
        plsc.subcore_barrier()

        def _drain(t, carry):
            r0 = pl.multiple_of(base_r + t * _K, 8)
            pltpu.sync_copy(acc.at[pl.ds(r0, _K)], rows0)
            pltpu.sync_copy(rows0, psum.at[p, c, pl.ds(r0, _K)])
            return carry

        lax.fori_loop(0, _RPT // _K, _drain, 0)


@functools.partial(jax.jit, static_argnums=())
def _agg(src0, dst0, src1, dst1, src2, dst2, wh0, wh1, wh2):
    mesh = plsc.VectorSubcoreMesh(core_axis_name="c", subcore_axis_name="s",
                                  num_cores=_NC, num_subcores=_NS)
    kern = pl.kernel(
        _agg_body,
        out_type=jax.ShapeDtypeStruct((6, _NC, _NP, _D), jnp.float32),
        mesh=mesh,
        scratch_types=[
            pltpu.VMEM_SHARED((_NP, _D), jnp.float32),
            pltpu.VMEM((_K,), jnp.int32),
            pltpu.VMEM((_K,), jnp.int32),
            pltpu.VMEM((_K,), jnp.int32),
            pltpu.VMEM((_K,), jnp.int32),
            pltpu.VMEM((_K, _D), jnp.float32),
            pltpu.VMEM((_K, _D), jnp.float32),
            pltpu.VMEM((_ZR, _D), jnp.float32),
            pltpu.SemaphoreType.DMA,
            pltpu.SemaphoreType.DMA,
        ],
    )
    return kern(src0, dst0, src1, dst1, src2, dst2, wh0, wh1, wh2)


def _combine_body(ps_ref, o_ref):
    acc = jnp.zeros(o_ref.shape, jnp.float32)
    for e in range(3):
        sm = ps_ref[e, 0] + ps_ref[e, 1]
        cn = ps_ref[3 + e, 0, :, 0:1] + ps_ref[3 + e, 1, :, 0:1]
        acc = acc + sm / jnp.maximum(cn, 1.0)
    o_ref[...] = jnp.maximum(acc, 0.0)


def _combine(psum):
    BM = 1000
    grid = (_N // BM,)
    return pl.pallas_call(
        _combine_body,
        grid=grid,
        in_specs=[pl.BlockSpec((6, _NC, BM, _D), lambda i: (0, 0, i, 0))],
        out_specs=pl.BlockSpec((BM, _D), lambda i: (i, 0)),
        out_shape=jax.ShapeDtypeStruct((_N, _D), jnp.float32),
    )(psum)


def kernel(features, edge_index_e0, edge_index_e1, edge_index_e2,
           W_e0, b_e0, W_e1, b_e1, W_e2, b_e2):
    wh0, wh1, wh2 = _matmul(features, W_e0, b_e0, W_e1, b_e1, W_e2, b_e2)
    psum = _agg(edge_index_e0[0], edge_index_e0[1],
                edge_index_e1[0], edge_index_e1[1],
                edge_index_e2[0], edge_index_e2[1],
                wh0, wh1, wh2)
    return _combine(psum)


# speedup vs baseline: 6.8643x; 1.1396x over previous
import functools

import jax
import jax.numpy as jnp
from jax import lax
from jax.experimental import pallas as pl
from jax.experimental.pallas import tpu as pltpu
from jax.experimental.pallas import tpu_sc as plsc

_N = 10000
_E = 320000
_D = 128
_NC = 2
_NS = 16
_NW = _NC * _NS
_EPT = _E // _NW
_K = 80
_NCHUNK = _EPT // _K
_NP = 10240
_RPT = _NP // _NS
_ZR = 8


def _matmul_body(f_ref, w0, b0, w1, b1, w2, b2, o0, o1, o2):
    f = f_ref[...]
    o0[...] = jnp.dot(f, w0[...], preferred_element_type=jnp.float32) + b0[...]
    o1[...] = jnp.dot(f, w1[...], preferred_element_type=jnp.float32) + b1[...]
    o2[...] = jnp.dot(f, w2[...], preferred_element_type=jnp.float32) + b2[...]


def _matmul(features, W0, b0, W1, b1, W2, b2):
    BM = 2000
    grid = (_N // BM,)
    wspec = pl.BlockSpec((_D, _D), lambda i: (0, 0))
    bspec = pl.BlockSpec((1, _D), lambda i: (0, 0))
    fspec = pl.BlockSpec((BM, _D), lambda i: (i, 0))
    return pl.pallas_call(
        _matmul_body,
        grid=grid,
        in_specs=[fspec, wspec, bspec, wspec, bspec, wspec, bspec],
        out_specs=[fspec, fspec, fspec],
        out_shape=[jax.ShapeDtypeStruct((_N, _D), jnp.float32)] * 3,
    )(features, W0, b0.reshape(1, _D), W1, b1.reshape(1, _D), W2, b2.reshape(1, _D))


def _agg_body(src0, dst0, src1, dst1, src2, dst2, wh0, wh1, wh2,
              psum,
              acc, idx_s0, idx_s1, idx_d0, idx_d1, rows0, rows1, ztmp,
              sem0, sem1, isem0, isem1):
    c = lax.axis_index("c")
    s = lax.axis_index("s")
    wid = c * _NS + s
    base_r = s * _RPT
    ebase = wid * _EPT
    idx_s = (idx_s0, idx_s1)
    idx_d = (idx_d0, idx_d1)
    rows = (rows0, rows1)
    gsem = (sem0, sem1)
    isem = (isem0, isem1)

    def _fill_z(r, carry):
        for j in range(_D // 16):
            ztmp[r, pl.ds(16 * j, 16)] = jnp.zeros((16,), jnp.float32)
        return carry

    lax.fori_loop(0, _ZR, _fill_z, 0)

    def _fill_ones(r, carry):
        for j in range(_D // 16):
            rows1[r, pl.ds(16 * j, 16)] = jnp.ones((16,), jnp.float32)
        return carry

    passes = ((src0, dst0, wh0, True), (src1, dst1, wh1, True),
              (src2, dst2, wh2, True), (src0, dst0, wh0, False),
              (src1, dst1, wh1, False), (src2, dst2, wh2, False))
    for p, (src_ref, dst_ref, wh_ref, is_sum) in enumerate(passes):
        if p == 3:
            lax.fori_loop(0, _K, _fill_ones, 0)

        def _zero(t, carry):
            r0 = pl.multiple_of(base_r + t * _ZR, 8)
            pltpu.sync_copy(ztmp, acc.at[pl.ds(r0, _ZR)])
            return carry

        lax.fori_loop(0, _RPT // _ZR, _zero, 0)
        plsc.subcore_barrier()

        if is_sum:
            for b in range(2):
                off = ebase + b * _K
                pltpu.sync_copy(src_ref.at[pl.ds(off, _K)], idx_s[b])
                pltpu.sync_copy(dst_ref.at[pl.ds(off, _K)], idx_d[b])
                pltpu.async_copy(wh_ref.at[idx_s[b]], rows[b], gsem[b])

            def _pair(jj, carry):
                for b in range(2):
                    pltpu.make_async_copy(wh_ref.at[pl.ds(0, _K)], rows[b],
                                          gsem[b]).wait()
                    pltpu.sync_copy(rows[b], acc.at[idx_d[b]], add=True)
                    off = ebase + (2 * jj + b) * _K
                    pltpu.async_copy(src_ref.at[pl.ds(off, _K)], idx_s[b],
                                     isem[b])
                    pltpu.async_copy(dst_ref.at[pl.ds(off, _K)], idx_d[b],
                                     isem[b])
                for b in range(2):
                    pltpu.make_async_copy(src_ref.at[pl.ds(0, _K)], idx_s[b],
                                          isem[b]).wait()
                    pltpu.make_async_copy(dst_ref.at[pl.ds(0, _K)], idx_d[b],
                                          isem[b]).wait()
                    pltpu.async_copy(wh_ref.at[idx_s[b]], rows[b], gsem[b])
                return carry

            lax.fori_loop(1, (_NCHUNK - 1) // 2, _pair, 0)
            for b in range(2):
                pltpu.make_async_copy(wh_ref.at[pl.ds(0, _K)], rows[b],
                                      gsem[b]).wait()
                pltpu.sync_copy(rows[b], acc.at[idx_d[b]], add=True)
            off = ebase + (_NCHUNK - 1) * _K
            pltpu.sync_copy(src_ref.at[pl.ds(off, _K)], idx_s0)
            pltpu.sync_copy(dst_ref.at[pl.ds(off, _K)], idx_d0)
            pltpu.async_copy(wh_ref.at[idx_s0], rows0, sem0).wait()
            pltpu.sync_copy(rows0, acc.at[idx_d0], add=True)
        else:
            for b in range(2):
                off = ebase + b * _K
                pltpu.async_copy(dst_ref.at[pl.ds(off, _K)], idx_d[b], gsem[b])

            def _chunk(jj, carry):
                for b in range(2):
                    pltpu.make_async_copy(dst_ref.at[pl.ds(0, _K)], idx_d[b],
                                          gsem[b]).wait()
                    pltpu.sync_copy(rows1, acc.at[idx_d[b]], add=True)
                    off = ebase + (2 * jj + b) * _K
                    pltpu.async_copy(dst_ref.at[pl.ds(off, _K)], idx_d[b],
                                     gsem[b])
                return carry

            lax.fori_loop(1, (_NCHUNK - 1) // 2, _chunk, 0)
            for b in range(2):
                pltpu.make_async_copy(dst_ref.at[pl.ds(0, _K)], idx_d[b],
                                      gsem[b]).wait()
                pltpu.sync_copy(rows1, acc.at[idx_d[b]], add=True)
            off = ebase + (_NCHUNK - 1) * _K
            pltpu.sync_copy(dst_ref.at[pl.ds(off, _K)], idx_d0)
            pltpu.sync_copy(rows1, acc.at[idx_d0], add=True)
        plsc.subcore_barrier()

        def _drain(t, carry):
            r0 = pl.multiple_of(base_r + t * _K, 8)
            pltpu.sync_copy(acc.at[pl.ds(r0, _K)], rows0)
            pltpu.sync_copy(rows0, psum.at[p, c, pl.ds(r0, _K)])
            return carry

        lax.fori_loop(0, _RPT // _K, _drain, 0)


@functools.partial(jax.jit, static_argnums=())
def _agg(src0, dst0, src1, dst1, src2, dst2, wh0, wh1, wh2):
    mesh = plsc.VectorSubcoreMesh(core_axis_name="c", subcore_axis_name="s",
                                  num_cores=_NC, num_subcores=_NS)
    kern = pl.kernel(
        _agg_body,
        out_type=jax.ShapeDtypeStruct((6, _NC, _NP, _D), jnp.float32),
        mesh=mesh,
        scratch_types=[
            pltpu.VMEM_SHARED((_NP, _D), jnp.float32),
            pltpu.VMEM((_K,), jnp.int32),
            pltpu.VMEM((_K,), jnp.int32),
            pltpu.VMEM((_K,), jnp.int32),
            pltpu.VMEM((_K,), jnp.int32),
            pltpu.VMEM((_K, _D), jnp.float32),
            pltpu.VMEM((_K, _D), jnp.float32),
            pltpu.VMEM((_ZR, _D), jnp.float32),
            pltpu.SemaphoreType.DMA,
            pltpu.SemaphoreType.DMA,
            pltpu.SemaphoreType.DMA,
            pltpu.SemaphoreType.DMA,
        ],
    )
    return kern(src0, dst0, src1, dst1, src2, dst2, wh0, wh1, wh2)


def _combine_body(ps_ref, o_ref):
    acc = jnp.zeros(o_ref.shape, jnp.float32)
    for e in range(3):
        sm = ps_ref[e, 0] + ps_ref[e, 1]
        cn = ps_ref[3 + e, 0, :, 0:1] + ps_ref[3 + e, 1, :, 0:1]
        acc = acc + sm / jnp.maximum(cn, 1.0)
    o_ref[...] = jnp.maximum(acc, 0.0)


def _combine(psum):
    BM = 1000
    grid = (_N // BM,)
    return pl.pallas_call(
        _combine_body,
        grid=grid,
        in_specs=[pl.BlockSpec((6, _NC, BM, _D), lambda i: (0, 0, i, 0))],
        out_specs=pl.BlockSpec((BM, _D), lambda i: (i, 0)),
        out_shape=jax.ShapeDtypeStruct((_N, _D), jnp.float32),
    )(psum)


def kernel(features, edge_index_e0, edge_index_e1, edge_index_e2,
           W_e0, b_e0, W_e1, b_e1, W_e2, b_e2):
    wh0, wh1, wh2 = _matmul(features, W_e0, b_e0, W_e1, b_e1, W_e2, b_e2)
    psum = _agg(edge_index_e0[0], edge_index_e0[1],
                edge_index_e1[0], edge_index_e1[1],
                edge_index_e2[0], edge_index_e2[1],
                wh0, wh1, wh2)
    return _combine(psum)
